# baseline (speedup 1.0000x reference)
import functools

import jax
import jax.numpy as jnp
from jax import lax
from jax.experimental import pallas as pl
from jax.experimental.pallas import tpu as pltpu
from jax.experimental.pallas import tpu_sc as plsc

_N = 51200
_E = 819200
_CHUNK = 128
_ROWS = _E // _CHUNK
_NSUB = 16
_NCORE = 2
_SEG = _N // _NSUB
_AROWS_T = _ROWS // _NSUB
_PROWS_T = _ROWS // (2 * _NSUB)
_RB = 8
_SB = 2048


def _enc_body(pos_ref, w1, b1, w2, b2, w3lo, b3lo, w3hi, b3hi, perm, ones_row,
              xlo, xhi, ptab):
    p = pos_ref[...]
    nsb = p.shape[0] // _SB
    pp = jnp.concatenate(
        [jnp.concatenate([p[sb * _SB + a * 256: sb * _SB + (a + 1) * 256]
                          for a in range(8)], axis=1)
         for sb in range(nsb)], axis=0)
    h = jnp.tanh(jnp.dot(pp, w1[...], preferred_element_type=jnp.float32)
                 + b1[...])
    h = jnp.tanh(jnp.dot(h, w2[...], preferred_element_type=jnp.float32)
                 + b2[...])
    xlo[...] = jnp.tanh(
        jnp.dot(h, w3lo[...], preferred_element_type=jnp.float32) + b3lo[...])
    xhi[...] = jnp.tanh(
        jnp.dot(h, w3hi[...], preferred_element_type=jnp.float32) + b3hi[...])
    ptab[...] = jnp.dot(pp, perm[...],
                        preferred_element_type=jnp.float32) + ones_row[...]


def _full(shape):
    return pl.BlockSpec(shape, lambda i: tuple(0 for _ in shape))


def _encode(pos, W1, b1, W2, b2, W3, b3):
    eye8 = jnp.eye(8, dtype=jnp.float32)
    w1 = jnp.kron(eye8, W1)
    w2 = jnp.kron(eye8, W2)
    w3lo = jnp.kron(eye8, W3[:, :16])
    w3hi = jnp.kron(eye8, W3[:, 16:])
    b1p = jnp.tile(b1, 8).reshape(1, 128)
    b2p = jnp.tile(b2, 8).reshape(1, 128)
    b3lo = jnp.tile(b3[:16], 8).reshape(1, 128)
    b3hi = jnp.tile(b3[16:], 8).reshape(1, 128)
    perm = jnp.kron(eye8, jnp.eye(3, 16, dtype=jnp.float32))
    ones_row = jnp.tile(jnp.eye(1, 16, k=3, dtype=jnp.float32)[0],
                        8).reshape(1, 128)
    nsb = 5
    blk = nsb * _SB
    return pl.pallas_call(
        _enc_body,
        grid=(_N // blk,),
        in_specs=[
            pl.BlockSpec((blk, 3), lambda i: (i, 0)),
            _full((24, 128)), _full((1, 128)),
            _full((128, 128)), _full((1, 128)),
            _full((128, 128)), _full((1, 128)),
            _full((128, 128)), _full((1, 128)),
            _full((24, 128)), _full((1, 128)),
        ],
        out_specs=[pl.BlockSpec((blk // 8, 128), lambda i: (i, 0))] * 3,
        out_shape=[jax.ShapeDtypeStruct((_N // 8, 128), jnp.float32)] * 3,
    )(pos, w1, b1p, w2, b2p, w3lo, b3lo, w3hi, b3hi, perm, ones_row)


def _ring(tab, acc, eir, sidx, didx, rowbuf, gsems, ssems, isem, jsem,
          row0, nb):

    def fire_idx(k, poff):
        src_rows = pl.ds(row0 + k * _RB, _RB)
        dst_rows = pl.ds(poff, _RB)
        pltpu.async_copy(eir.at[0, src_rows], sidx.at[dst_rows], isem)
        pltpu.async_copy(eir.at[1, src_rows], didx.at[dst_rows], jsem)

    def wait_idx():
        pltpu.make_async_copy(eir.at[0, pl.ds(0, _RB)],
                              sidx.at[pl.ds(0, _RB)], isem).wait()
        pltpu.make_async_copy(eir.at[0, pl.ds(0, _RB)],
                              didx.at[pl.ds(0, _RB)], jsem).wait()

    def fire_g(idxrow, r):
        pltpu.async_copy(tab.at[sidx.at[idxrow]], rowbuf.at[r], gsems[r])

    def fire_s(idxrow, r):
        pltpu.async_copy(rowbuf.at[r], acc.at[didx.at[idxrow]],
                         ssems[r], add=True)

    def wait_g(r):
        pltpu.make_async_copy(tab.at[pl.ds(0, _CHUNK)], rowbuf.at[r],
                              gsems[r]).wait()

    def wait_s(r):
        pltpu.make_async_copy(tab.at[pl.ds(0, _CHUNK)], rowbuf.at[r],
                              ssems[r]).wait()

    def body(k, p8, first, last):
        np8 = _RB - p8
        if not last:
            fire_idx(k + 1, np8)
        for r in range(4):
            wait_g(r)
            fire_s(p8 + r, r)
            if not first:
                wait_s(r + 4)
            fire_g(p8 + r + 4, r + 4)
        if not last:
            wait_idx()
        for r in range(4, _RB):
            wait_g(r)
            fire_s(p8 + r, r)
            wait_s(r - 4)
            if not last:
                fire_g(np8 + r - 4, r - 4)

    pltpu.sync_copy(eir.at[0, pl.ds(row0, _RB)], sidx.at[pl.ds(0, _RB)])
    pltpu.sync_copy(eir.at[1, pl.ds(row0, _RB)], didx.at[pl.ds(0, _RB)])
    for r in range(4):
        fire_g(r, r)
    body(0, 0, first=True, last=False)

    def mid(k, carry):
        body(k, (k % 2) * _RB, first=False, last=False)
        return carry
    lax.fori_loop(1, nb - 1, mid, 0)

    body(nb - 1, ((nb - 1) % 2) * _RB, first=False, last=True)
    for r in range(4, _RB):
        wait_s(r)


def _sc_body(xlo, xhi, ptab, eir, outX0, outX1, outP0, outP1,
             sidx, didx, rowbuf, accA, accP, *sems):
    gsems = sems[:_RB]
    ssems = sems[_RB:2 * _RB]
    isem = sems[2 * _RB]
    jsem = sems[2 * _RB + 1]
    c = lax.axis_index("c")
    s = lax.axis_index("s")
    sl = pl.ds(s * _SEG, _SEG)

    @pl.when(c == 0)
    def _():
        pltpu.sync_copy(xlo.at[sl], accA.at[sl])

    @pl.when(c == 1)
    def _():
        pltpu.sync_copy(xhi.at[sl], accA.at[sl])

    pltpu.sync_copy(ptab.at[sl], accP.at[sl])
    plsc.subcore_barrier()

    @pl.when(c == 0)
    def _():
        _ring(xlo, accA, eir, sidx, didx, rowbuf, gsems, ssems, isem, jsem,
              s * _AROWS_T, _AROWS_T // _RB)

    @pl.when(c == 1)
    def _():
        _ring(xhi, accA, eir, sidx, didx, rowbuf, gsems, ssems, isem, jsem,
              s * _AROWS_T, _AROWS_T // _RB)

    _ring(ptab, accP, eir, sidx, didx, rowbuf, gsems, ssems, isem, jsem,
          c * (_ROWS // 2) + s * _PROWS_T, _PROWS_T // _RB)

    plsc.subcore_barrier()

    @pl.when(c == 0)
    def _():
        pltpu.sync_copy(accA.at[sl], outX0.at[sl])
        pltpu.sync_copy(accP.at[sl], outP0.at[sl])

    @pl.when(c == 1)
    def _():
        pltpu.sync_copy(accA.at[sl], outX1.at[sl])
        pltpu.sync_copy(accP.at[sl], outP1.at[sl])


_sc_segsum = functools.partial(
    pl.kernel,
    out_type=[jax.ShapeDtypeStruct((_N, 16), jnp.float32)] * 4,
    mesh=plsc.VectorSubcoreMesh(core_axis_name="c", subcore_axis_name="s",
                                num_cores=_NCORE, num_subcores=_NSUB),
    scratch_types=(
        [
            pltpu.VMEM((2 * _RB, _CHUNK), jnp.int32),
            pltpu.VMEM((2 * _RB, _CHUNK), jnp.int32),
            pltpu.VMEM((_RB, _CHUNK, 16), jnp.float32),
            pltpu.VMEM_SHARED((_N, 16), jnp.float32),
            pltpu.VMEM_SHARED((_N, 16), jnp.float32),
        ]
        + [pltpu.SemaphoreType.DMA] * (2 * _RB + 2)
    ),
    compiler_params=pltpu.CompilerParams(use_tc_tiling_on_sc=False),
)(_sc_body)


def _pool_body(x0, x1, p0, p1, pt, out):
    PT = pt[...]
    P = p0[...] + p1[...] - PT
    lane = lax.broadcasted_iota(jnp.int32, (256, 128), 1)
    row = lax.broadcasted_iota(jnp.int32, (128, 128), 0)
    lane128 = lane[:128]
    sel = (lane % 16) == 3
    rcp = jnp.where(sel, 1.0 / P, 0.0)
    bmat = jnp.where(((row % 16) == 3) & ((row // 16) == (lane128 // 16)),
                     1.0, 0.0)
    C = jnp.dot(rcp, bmat, preferred_element_type=jnp.float32)
    vx0 = x0[...] * C
    vx1 = x1[...] * C
    vp = P * C - PT
    sx0 = jnp.sum(vx0, axis=0, keepdims=True)
    sx1 = jnp.sum(vx1, axis=0, keepdims=True)
    sp = jnp.sum(vp, axis=0, keepdims=True)
    rows = []
    for h in range(2):
        half = (row >= 64 * h) & (row < 64 * (h + 1))
        f0 = jnp.where(half & ((row % 16) == lane128) & (lane128 < 16),
                       1.0, 0.0)
        f1 = jnp.where(half & ((row % 16) == lane128 - 16)
                       & (lane128 >= 16) & (lane128 < 32), 1.0, 0.0)
        f2 = jnp.where(half & ((row % 16) == lane128 - 32)
                       & (lane128 >= 32) & (lane128 < 48), 1.0, 0.0)
        rows.append(jnp.dot(sx0, f0, preferred_element_type=jnp.float32)
                    + jnp.dot(sx1, f1, preferred_element_type=jnp.float32)
                    + jnp.dot(sp, f2, preferred_element_type=jnp.float32))
    out[...] = (jnp.concatenate(rows, axis=0)
                * (1.0 / 1024.0)).reshape(1, 2, 128)


def _pool(x0, x1, p0, p1, pt):
    nblk = _N // _SB
    return pl.pallas_call(
        _pool_body,
        grid=(nblk,),
        in_specs=[pl.BlockSpec((256, 128), lambda i: (i, 0))] * 5,
        out_specs=pl.BlockSpec((1, 2, 128), lambda i: (i, 0, 0)),
        out_shape=jax.ShapeDtypeStruct((nblk, 2, 128), jnp.float32),
    )(x0, x1, p0, p1, pt)


def kernel(pos, edge_index, W_enc1, b_enc1, W_enc2, b_enc2, W_enc3, b_enc3,
           W_loc, b_loc, W_glob, b_glob, W_dec1, b_dec1, W_dec2, b_dec2,
           W_dec3, b_dec3):
    xlo, xhi, ptab = _encode(pos, W_enc1, b_enc1, W_enc2, b_enc2, W_enc3,
                             b_enc3)
    nloc = edge_index % _SB
    pk = (edge_index - nloc) + (nloc % 256) * 8 + nloc // 256
    eir = pk.reshape(2, _ROWS, _CHUNK)
    outX0, outX1, outP0, outP1 = _sc_segsum(
        xlo.reshape(_N, 16), xhi.reshape(_N, 16), ptab.reshape(_N, 16), eir)
    pooled = _pool(outX0.reshape(_N // 8, 128), outX1.reshape(_N // 8, 128),
                   outP0.reshape(_N // 8, 128), outP1.reshape(_N // 8, 128),
                   ptab).reshape(_N // 1024, 128)
    h = jnp.concatenate([pooled[:, :32], pooled[:, 32:35]], axis=1) @ W_loc + b_loc
    h = h @ W_glob + b_glob
    h = jnp.tanh(h @ W_dec1 + b_dec1)
    h = jnp.tanh(h @ W_dec2 + b_dec2)
    return h @ W_dec3 + b_dec3


# speedup vs baseline: 1.0995x; 1.0995x over previous
import functools

import jax
import jax.numpy as jnp
from jax import lax
from jax.experimental import pallas as pl
from jax.experimental.pallas import tpu as pltpu
from jax.experimental.pallas import tpu_sc as plsc

_N = 51200
_E = 819200
_CHUNK = 128
_ROWS = _E // _CHUNK
_NSUB = 16
_NCORE = 2
_SEG = _N // _NSUB
_AROWS_T = _ROWS // _NSUB
_PROWS_T = _ROWS // (2 * _NSUB)
_RB = 8
_SB = 2048


def _enc_body(pos_ref, w1, b1, w2, b2, w3lo, b3lo, w3hi, b3hi, perm, ones_row,
              xlo, xhi, ptab):
    p = pos_ref[...]
    nsb = p.shape[0] // _SB
    pp = jnp.concatenate(
        [jnp.concatenate([p[sb * _SB + a * 256: sb * _SB + (a + 1) * 256]
                          for a in range(8)], axis=1)
         for sb in range(nsb)], axis=0)
    h = jnp.tanh(jnp.dot(pp, w1[...], preferred_element_type=jnp.float32)
                 + b1[...])
    h = jnp.tanh(jnp.dot(h, w2[...], preferred_element_type=jnp.float32)
                 + b2[...])
    xlo[...] = jnp.tanh(
        jnp.dot(h, w3lo[...], preferred_element_type=jnp.float32) + b3lo[...])
    xhi[...] = jnp.tanh(
        jnp.dot(h, w3hi[...], preferred_element_type=jnp.float32) + b3hi[...])
    ptab[...] = jnp.dot(pp, perm[...],
                        preferred_element_type=jnp.float32) + ones_row[...]


def _full(shape):
    return pl.BlockSpec(shape, lambda i: tuple(0 for _ in shape))


def _encode(pos, W1, b1, W2, b2, W3, b3):
    eye8 = jnp.eye(8, dtype=jnp.float32)
    w1 = jnp.kron(eye8, W1)
    w2 = jnp.kron(eye8, W2)
    w3lo = jnp.kron(eye8, W3[:, :16])
    w3hi = jnp.kron(eye8, W3[:, 16:])
    b1p = jnp.tile(b1, 8).reshape(1, 128)
    b2p = jnp.tile(b2, 8).reshape(1, 128)
    b3lo = jnp.tile(b3[:16], 8).reshape(1, 128)
    b3hi = jnp.tile(b3[16:], 8).reshape(1, 128)
    perm = jnp.kron(eye8, jnp.eye(3, 16, dtype=jnp.float32))
    ones_row = jnp.tile(jnp.eye(1, 16, k=3, dtype=jnp.float32)[0],
                        8).reshape(1, 128)
    nsb = 5
    blk = nsb * _SB
    return pl.pallas_call(
        _enc_body,
        grid=(_N // blk,),
        in_specs=[
            pl.BlockSpec((blk, 3), lambda i: (i, 0)),
            _full((24, 128)), _full((1, 128)),
            _full((128, 128)), _full((1, 128)),
            _full((128, 128)), _full((1, 128)),
            _full((128, 128)), _full((1, 128)),
            _full((24, 128)), _full((1, 128)),
        ],
        out_specs=[pl.BlockSpec((blk // 8, 128), lambda i: (i, 0))] * 3,
        out_shape=[jax.ShapeDtypeStruct((_N // 8, 128), jnp.float32)] * 3,
    )(pos, w1, b1p, w2, b2p, w3lo, b3lo, w3hi, b3hi, perm, ones_row)


def _ring(tab, acc, eir, sidx, didx, rowbuf, gsems, ssems, isem, jsem,
          row0, nb):

    def fire_idx(k, poff):
        src_rows = pl.ds(row0 + k * _RB, _RB)
        dst_rows = pl.ds(poff, _RB)
        pltpu.async_copy(eir.at[0, src_rows], sidx.at[dst_rows], isem)
        pltpu.async_copy(eir.at[1, src_rows], didx.at[dst_rows], jsem)

    def wait_idx():
        pltpu.make_async_copy(eir.at[0, pl.ds(0, _RB)],
                              sidx.at[pl.ds(0, _RB)], isem).wait()
        pltpu.make_async_copy(eir.at[0, pl.ds(0, _RB)],
                              didx.at[pl.ds(0, _RB)], jsem).wait()

    def fire_g(idxrow, r):
        pltpu.async_copy(tab.at[sidx.at[idxrow]], rowbuf.at[r], gsems[r])

    def fire_s(idxrow, r):
        pltpu.async_copy(rowbuf.at[r], acc.at[didx.at[idxrow]],
                         ssems[r], add=True)

    def wait_g(r):
        pltpu.make_async_copy(tab.at[pl.ds(0, _CHUNK)], rowbuf.at[r],
                              gsems[r]).wait()

    def wait_s(r):
        pltpu.make_async_copy(tab.at[pl.ds(0, _CHUNK)], rowbuf.at[r],
                              ssems[r]).wait()

    def body(k, p8, first, last):
        np8 = _RB - p8
        if not last:
            fire_idx(k + 1, np8)
        for r in range(4):
            wait_g(r)
            fire_s(p8 + r, r)
            if not first:
                wait_s(r + 4)
            fire_g(p8 + r + 4, r + 4)
        if not last:
            wait_idx()
        for r in range(4, _RB):
            wait_g(r)
            fire_s(p8 + r, r)
            wait_s(r - 4)
            if not last:
                fire_g(np8 + r - 4, r - 4)

    pltpu.sync_copy(eir.at[0, pl.ds(row0, _RB)], sidx.at[pl.ds(0, _RB)])
    pltpu.sync_copy(eir.at[1, pl.ds(row0, _RB)], didx.at[pl.ds(0, _RB)])
    for r in range(4):
        fire_g(r, r)
    body(0, 0, first=True, last=False)

    def mid(k, carry):
        body(k, (k % 2) * _RB, first=False, last=False)
        return carry
    lax.fori_loop(1, nb - 1, mid, 0)

    body(nb - 1, ((nb - 1) % 2) * _RB, first=False, last=True)
    for r in range(4, _RB):
        wait_s(r)


def _sc_body(xlo, xhi, ptab, eir, outX0, outX1, outP0, outP1,
             sidx, didx, rowbuf, accA, accP, *sems):
    gsems = sems[:_RB]
    ssems = sems[_RB:2 * _RB]
    isem = sems[2 * _RB]
    jsem = sems[2 * _RB + 1]
    c = lax.axis_index("c")
    s = lax.axis_index("s")
    sl = pl.ds(s * _SEG, _SEG)

    @pl.when(c == 0)
    def _():
        pltpu.sync_copy(xlo.at[sl], accA.at[sl])

    @pl.when(c == 1)
    def _():
        pltpu.sync_copy(xhi.at[sl], accA.at[sl])

    pltpu.sync_copy(ptab.at[sl], accP.at[sl])
    plsc.subcore_barrier()

    @pl.when(c == 0)
    def _():
        _ring(xlo, accA, eir, sidx, didx, rowbuf, gsems, ssems, isem, jsem,
              s * _AROWS_T, _AROWS_T // _RB)

    @pl.when(c == 1)
    def _():
        _ring(xhi, accA, eir, sidx, didx, rowbuf, gsems, ssems, isem, jsem,
              s * _AROWS_T, _AROWS_T // _RB)

    _ring(ptab, accP, eir, sidx, didx, rowbuf, gsems, ssems, isem, jsem,
          c * (_ROWS // 2) + s * _PROWS_T, _PROWS_T // _RB)

    plsc.subcore_barrier()

    @pl.when(c == 0)
    def _():
        pltpu.sync_copy(accA.at[sl], outX0.at[sl])
        pltpu.sync_copy(accP.at[sl], outP0.at[sl])

    @pl.when(c == 1)
    def _():
        pltpu.sync_copy(accA.at[sl], outX1.at[sl])
        pltpu.sync_copy(accP.at[sl], outP1.at[sl])


_sc_segsum = functools.partial(
    pl.kernel,
    out_type=[jax.ShapeDtypeStruct((_N, 16), jnp.float32)] * 4,
    mesh=plsc.VectorSubcoreMesh(core_axis_name="c", subcore_axis_name="s",
                                num_cores=_NCORE, num_subcores=_NSUB),
    scratch_types=(
        [
            pltpu.VMEM((2 * _RB, _CHUNK), jnp.int32),
            pltpu.VMEM((2 * _RB, _CHUNK), jnp.int32),
            pltpu.VMEM((_RB, _CHUNK, 16), jnp.float32),
            pltpu.VMEM_SHARED((_N, 16), jnp.float32),
            pltpu.VMEM_SHARED((_N, 16), jnp.float32),
        ]
        + [pltpu.SemaphoreType.DMA] * (2 * _RB + 2)
    ),
    compiler_params=pltpu.CompilerParams(use_tc_tiling_on_sc=False),
)(_sc_body)


def _pool_body(x0, x1, p0, p1, pt, out):
    PT = pt[...]
    P = p0[...] + p1[...] - PT
    lane = lax.broadcasted_iota(jnp.int32, (256, 128), 1)
    row = lax.broadcasted_iota(jnp.int32, (128, 128), 0)
    lane128 = lane[:128]
    sel = (lane % 16) == 3
    rcp = jnp.where(sel, 1.0 / P, 0.0)
    bmat = jnp.where(((row % 16) == 3) & ((row // 16) == (lane128 // 16)),
                     1.0, 0.0)
    C = jnp.dot(rcp, bmat, preferred_element_type=jnp.float32)
    vx0 = x0[...] * C
    vx1 = x1[...] * C
    vp = P * C - PT
    sx0 = jnp.sum(vx0, axis=0, keepdims=True)
    sx1 = jnp.sum(vx1, axis=0, keepdims=True)
    sp = jnp.sum(vp, axis=0, keepdims=True)
    rows = []
    for h in range(2):
        half = (row >= 64 * h) & (row < 64 * (h + 1))
        f0 = jnp.where(half & ((row % 16) == lane128) & (lane128 < 16),
                       1.0, 0.0)
        f1 = jnp.where(half & ((row % 16) == lane128 - 16)
                       & (lane128 >= 16) & (lane128 < 32), 1.0, 0.0)
        f2 = jnp.where(half & ((row % 16) == lane128 - 32)
                       & (lane128 >= 32) & (lane128 < 48), 1.0, 0.0)
        rows.append(jnp.dot(sx0, f0, preferred_element_type=jnp.float32)
                    + jnp.dot(sx1, f1, preferred_element_type=jnp.float32)
                    + jnp.dot(sp, f2, preferred_element_type=jnp.float32))
    out[...] = (jnp.concatenate(rows, axis=0)
                * (1.0 / 1024.0)).reshape(1, 2, 128)


def _pool(x0, x1, p0, p1, pt):
    nblk = _N // _SB
    return pl.pallas_call(
        _pool_body,
        grid=(nblk,),
        in_specs=[pl.BlockSpec((256, 128), lambda i: (i, 0))] * 5,
        out_specs=pl.BlockSpec((1, 2, 128), lambda i: (i, 0, 0)),
        out_shape=jax.ShapeDtypeStruct((nblk, 2, 128), jnp.float32),
    )(x0, x1, p0, p1, pt)


def kernel(pos, edge_index, W_enc1, b_enc1, W_enc2, b_enc2, W_enc3, b_enc3,
           W_loc, b_loc, W_glob, b_glob, W_dec1, b_dec1, W_dec2, b_dec2,
           W_dec3, b_dec3):
    xlo, xhi, ptab = _encode(pos, W_enc1, b_enc1, W_enc2, b_enc2, W_enc3,
                             b_enc3)
    eif = edge_index.reshape(2, _ROWS, _CHUNK)
    nloc = eif % _SB
    eir = (eif - nloc) + (nloc % 256) * 8 + nloc // 256
    outX0, outX1, outP0, outP1 = _sc_segsum(
        xlo.reshape(_N, 16), xhi.reshape(_N, 16), ptab.reshape(_N, 16), eir)
    pooled = _pool(outX0.reshape(_N // 8, 128), outX1.reshape(_N // 8, 128),
                   outP0.reshape(_N // 8, 128), outP1.reshape(_N // 8, 128),
                   ptab).reshape(_N // 1024, 128)
    h = jnp.concatenate([pooled[:, :32], pooled[:, 32:35]], axis=1) @ W_loc + b_loc
    h = h @ W_glob + b_glob
    h = jnp.tanh(h @ W_dec1 + b_dec1)
    h = jnp.tanh(h @ W_dec2 + b_dec2)
    return h @ W_dec3 + b_dec3
